# baseline (speedup 1.0000x reference)
import jax
import jax.numpy as jnp
from jax.experimental import pallas as pl
from jax.experimental.pallas import tpu as pltpu

_SCALE = 0.5
_EPS = 1e-05

_ROWS = 8192
_COLS = 1024
_CHUNK = 512
_NSTEPS = _ROWS // _CHUNK


def _berhu_body(pred_ref, gt_ref, out_ref, dv_ref, s_ref, m_ref, c_ref,
                w_ref, b_ref, acc_ref):
    i = pl.program_id(0)

    @pl.when(i == 0)
    def _init():
        s_ref[...] = jnp.zeros_like(s_ref)
        m_ref[...] = jnp.zeros_like(m_ref)
        c_ref[...] = jnp.zeros_like(c_ref)

    p = pred_ref[...]
    g = gt_ref[...]
    valid = g > _EPS
    dv = jnp.where(valid, jnp.abs(p - g), 0.0)
    dv_ref[pl.ds(i * _CHUNK, _CHUNK), :] = dv
    s_ref[...] = s_ref[...] + dv
    m_ref[...] = jnp.maximum(m_ref[...], dv)
    c_ref[...] = c_ref[...] + jnp.where(valid, 1.0, 0.0)

    @pl.when(i == _NSTEPS - 1)
    def _finish():
        t = _SCALE * jnp.max(m_ref[...])
        denom = 2.0 * t + _EPS
        w_ref[...] = jnp.zeros_like(w_ref)
        b_ref[...] = jnp.zeros_like(b_ref)

        def loop(j, _):
            blk = dv_ref[pl.ds(j * _CHUNK, _CHUNK), :]
            q = jnp.maximum(blk - t, 0.0)
            w_ref[...] = w_ref[...] + q * q
            b_ref[...] = b_ref[...] + jnp.where(blk > t, blk, 0.0)
            return 0

        jax.lax.fori_loop(0, _NSTEPS, loop, 0)
        total = jnp.sum(s_ref[...]) + (
            jnp.sum(w_ref[...]) - _EPS * jnp.sum(b_ref[...])) / denom
        out_ref[0] = total / jnp.sum(c_ref[...])


def kernel(pred, gt):
    p2 = pred.reshape(_ROWS, _COLS)
    g2 = gt.reshape(_ROWS, _COLS)
    out = pl.pallas_call(
        _berhu_body,
        grid=(_NSTEPS,),
        in_specs=[
            pl.BlockSpec((_CHUNK, _COLS), lambda i: (i, 0)),
            pl.BlockSpec((_CHUNK, _COLS), lambda i: (i, 0)),
        ],
        out_specs=pl.BlockSpec(memory_space=pltpu.SMEM),
        out_shape=jax.ShapeDtypeStruct((1,), jnp.float32),
        scratch_shapes=[
            pltpu.VMEM((_ROWS, _COLS), jnp.float32),
            pltpu.VMEM((_CHUNK, _COLS), jnp.float32),
            pltpu.VMEM((_CHUNK, _COLS), jnp.float32),
            pltpu.VMEM((_CHUNK, _COLS), jnp.float32),
            pltpu.VMEM((_CHUNK, _COLS), jnp.float32),
            pltpu.VMEM((_CHUNK, _COLS), jnp.float32),
            pltpu.SMEM((4,), jnp.float32),
        ],
        compiler_params=pltpu.CompilerParams(
            vmem_limit_bytes=56 * 1024 * 1024,
        ),
    )(p2, g2)
    return out[0]


# speedup vs baseline: 2.5101x; 2.5101x over previous
import jax
import jax.numpy as jnp
from jax.experimental import pallas as pl
from jax.experimental.pallas import tpu as pltpu

_SCALE = 0.5
_EPS = 1e-05

_B = 32
_H = 512
_W = 512
_NSTEPS = _B


def _berhu_body(pred_ref, gt_ref, out_ref, dv_ref, s_ref, m_ref, c_ref,
                w_ref, b_ref):
    i = pl.program_id(0)

    @pl.when(i == 0)
    def _init():
        s_ref[...] = jnp.zeros_like(s_ref)
        m_ref[...] = jnp.zeros_like(m_ref)
        c_ref[...] = jnp.zeros_like(c_ref)

    p = pred_ref[0, 0]
    g = gt_ref[0, 0]
    valid = g > _EPS
    dv = jnp.where(valid, jnp.abs(p - g), 0.0)
    dv_ref[i] = dv
    s_ref[...] = s_ref[...] + dv
    m_ref[...] = jnp.maximum(m_ref[...], dv)
    c_ref[...] = c_ref[...] + jnp.where(valid, 1.0, 0.0)

    @pl.when(i == _NSTEPS - 1)
    def _finish():
        t = _SCALE * jnp.max(m_ref[...])
        denom = 2.0 * t + _EPS
        w_ref[...] = jnp.zeros_like(w_ref)
        b_ref[...] = jnp.zeros_like(b_ref)

        def loop(j, _):
            blk = dv_ref[j]
            q = jnp.maximum(blk - t, 0.0)
            w_ref[...] = w_ref[...] + q * q
            b_ref[...] = b_ref[...] + jnp.where(blk > t, blk, 0.0)
            return 0

        jax.lax.fori_loop(0, _NSTEPS, loop, 0)
        total = jnp.sum(s_ref[...]) + (
            jnp.sum(w_ref[...]) - _EPS * jnp.sum(b_ref[...])) / denom
        out_ref[0] = total / jnp.sum(c_ref[...])


def kernel(pred, gt):
    out = pl.pallas_call(
        _berhu_body,
        grid=(_NSTEPS,),
        in_specs=[
            pl.BlockSpec((1, 1, _H, _W), lambda i: (i, 0, 0, 0)),
            pl.BlockSpec((1, 1, _H, _W), lambda i: (i, 0, 0, 0)),
        ],
        out_specs=pl.BlockSpec(memory_space=pltpu.SMEM),
        out_shape=jax.ShapeDtypeStruct((1,), jnp.float32),
        scratch_shapes=[
            pltpu.VMEM((_B, _H, _W), jnp.float32),
            pltpu.VMEM((_H, _W), jnp.float32),
            pltpu.VMEM((_H, _W), jnp.float32),
            pltpu.VMEM((_H, _W), jnp.float32),
            pltpu.VMEM((_H, _W), jnp.float32),
            pltpu.VMEM((_H, _W), jnp.float32),
        ],
        compiler_params=pltpu.CompilerParams(
            vmem_limit_bytes=56 * 1024 * 1024,
        ),
    )(pred, gt)
    return out[0]


# speedup vs baseline: 3.4722x; 1.3833x over previous
import jax
import jax.numpy as jnp
from jax.experimental import pallas as pl
from jax.experimental.pallas import tpu as pltpu

_SCALE = 0.5
_EPS = 1e-05

_B = 32
_H = 512
_W = 512
_K = 4
_NSTEPS = _B // _K


def _berhu_body(p0, p1, p2, p3, g0, g1, g2, g3, out_ref, dv_ref,
                s_ref, m_ref, c_ref, w_ref, b_ref):
    i = pl.program_id(0)

    @pl.when(i == 0)
    def _init():
        s_ref[...] = jnp.zeros_like(s_ref)
        m_ref[...] = jnp.zeros_like(m_ref)
        c_ref[...] = jnp.zeros_like(c_ref)

    s = s_ref[...]
    m = m_ref[...]
    c = c_ref[...]
    for k, (pr, gr) in enumerate(((p0, g0), (p1, g1), (p2, g2), (p3, g3))):
        p = pr[0, 0]
        g = gr[0, 0]
        valid = g > _EPS
        dv = jnp.where(valid, jnp.abs(p - g), 0.0)
        dv_ref[_K * i + k] = dv
        s = s + dv
        m = jnp.maximum(m, dv)
        c = c + jnp.where(valid, 1.0, 0.0)
    s_ref[...] = s
    m_ref[...] = m
    c_ref[...] = c

    @pl.when(i == _NSTEPS - 1)
    def _finish():
        t = _SCALE * jnp.max(m_ref[...])
        denom = 2.0 * t + _EPS
        w_ref[...] = jnp.zeros_like(w_ref)
        b_ref[...] = jnp.zeros_like(b_ref)

        def loop(j, _):
            blk = dv_ref[j]
            q = jnp.maximum(blk - t, 0.0)
            w_ref[...] = w_ref[...] + q * q
            b_ref[...] = b_ref[...] + jnp.where(blk > t, blk, 0.0)
            return 0

        jax.lax.fori_loop(0, _B, loop, 0)
        total = jnp.sum(s_ref[...]) + (
            jnp.sum(w_ref[...]) - _EPS * jnp.sum(b_ref[...])) / denom
        out_ref[0] = total / jnp.sum(c_ref[...])


def kernel(pred, gt):
    def spec(k):
        return pl.BlockSpec((1, 1, _H, _W), lambda i, k=k: (_K * i + k, 0, 0, 0))

    out = pl.pallas_call(
        _berhu_body,
        grid=(_NSTEPS,),
        in_specs=[spec(k) for k in range(_K)] * 2,
        out_specs=pl.BlockSpec(memory_space=pltpu.SMEM),
        out_shape=jax.ShapeDtypeStruct((1,), jnp.float32),
        scratch_shapes=[
            pltpu.VMEM((_B, _H, _W), jnp.float32),
            pltpu.VMEM((_H, _W), jnp.float32),
            pltpu.VMEM((_H, _W), jnp.float32),
            pltpu.VMEM((_H, _W), jnp.float32),
            pltpu.VMEM((_H, _W), jnp.float32),
            pltpu.VMEM((_H, _W), jnp.float32),
        ],
        compiler_params=pltpu.CompilerParams(
            vmem_limit_bytes=58 * 1024 * 1024,
        ),
    )(pred, pred, pred, pred, gt, gt, gt, gt)
    return out[0]
